# baseline (speedup 1.0000x reference)
import functools

import jax
import jax.numpy as jnp
from jax import lax
from jax.experimental import pallas as pl
from jax.experimental.pallas import tpu as pltpu
from jax.experimental.pallas import tpu_sc as plsc

B = 16384
K = 20
NC = 2
NS = 16
NW = NC * NS
BT = B // NW
C = 16
NCH = BT // C
VN_ROWS = C * K
IVW = 64
VN_DMAS = VN_ROWS // IVW

_mesh = plsc.VectorSubcoreMesh(core_axis_name="c", subcore_axis_name="s")

_buf_set = lambda: [
    pltpu.VMEM((C, 16), jnp.float32),
    pltpu.VMEM((C, 32), jnp.float32),
    pltpu.VMEM((C, 64), jnp.float32),
    pltpu.VMEM((C, 16), jnp.float32),
    pltpu.VMEM((C, 32), jnp.float32),
    pltpu.VMEM((C, 64), jnp.float32),
    pltpu.VMEM((VN_ROWS, 16), jnp.float32),
    pltpu.VMEM((VN_ROWS, 32), jnp.float32),
    pltpu.VMEM((VN_ROWS, 64), jnp.float32),
]


@functools.partial(
    pl.kernel,
    mesh=_mesh,
    compiler_params=pltpu.CompilerParams(use_tc_tiling_on_sc=False),
    out_type=[
        jax.ShapeDtypeStruct((B, 16), jnp.float32),
        jax.ShapeDtypeStruct((B, 16), jnp.float32),
    ],
    scratch_types=[
        pltpu.VMEM((NCH, C), jnp.int32),
        pltpu.VMEM((NCH, C), jnp.int32),
        pltpu.VMEM((NCH, C), jnp.int32),
        pltpu.VMEM((NCH, C), jnp.int32),
        pltpu.VMEM((NCH, C), jnp.int32),
        pltpu.VMEM((NCH, C), jnp.int32),
        pltpu.VMEM((NCH * VN_DMAS, IVW), jnp.int32),
        pltpu.VMEM((NCH * VN_DMAS, IVW), jnp.int32),
        pltpu.VMEM((NCH * VN_DMAS, IVW), jnp.int32),
        *_buf_set(),
        *_buf_set(),
        pltpu.VMEM((C, 16), jnp.float32),
        pltpu.VMEM((C, 16), jnp.float32),
        pltpu.SemaphoreType.DMA,
        pltpu.SemaphoreType.DMA,
    ],
)
def _sc_scores(W0, W1, W2, u0, u1, u2, p0, p1, p2, v0, v1, v2,
               score_o, neg_o,
               iu0, iu1, iu2, ip0, ip1, ip2, iv0, iv1, iv2,
               a_ub0, a_ub1, a_ub2, a_pb0, a_pb1, a_pb2, a_nb0, a_nb1, a_nb2,
               b_ub0, b_ub1, b_ub2, b_pb0, b_pb1, b_pb2, b_nb0, b_nb1, b_nb2,
               ps, pn, semA, semB):
    wid = lax.axis_index("s") * NC + lax.axis_index("c")
    seta = (a_ub0, a_ub1, a_ub2, a_pb0, a_pb1, a_pb2, a_nb0, a_nb1, a_nb2)
    setb = (b_ub0, b_ub1, b_ub2, b_pb0, b_pb1, b_pb2, b_nb0, b_nb1, b_nb2)

    hs = [
        pltpu.async_copy(u0.at[pl.ds(wid * NCH, NCH)], iu0, semA),
        pltpu.async_copy(u1.at[pl.ds(wid * NCH, NCH)], iu1, semA),
        pltpu.async_copy(u2.at[pl.ds(wid * NCH, NCH)], iu2, semA),
        pltpu.async_copy(p0.at[pl.ds(wid * NCH, NCH)], ip0, semA),
        pltpu.async_copy(p1.at[pl.ds(wid * NCH, NCH)], ip1, semA),
        pltpu.async_copy(p2.at[pl.ds(wid * NCH, NCH)], ip2, semA),
        pltpu.async_copy(v0.at[pl.ds(wid * NCH * VN_DMAS, NCH * VN_DMAS)], iv0, semA),
        pltpu.async_copy(v1.at[pl.ds(wid * NCH * VN_DMAS, NCH * VN_DMAS)], iv1, semA),
        pltpu.async_copy(v2.at[pl.ds(wid * NCH * VN_DMAS, NCH * VN_DMAS)], iv2, semA),
    ]
    for h in hs:
        h.wait()

    def descriptors(c, bufs, sem):
        ub0, ub1, ub2, pb0, pb1, pb2, nb0, nb1, nb2 = bufs
        ds = [
            pltpu.make_async_copy(W0.at[iu0.at[c]], ub0, sem),
            pltpu.make_async_copy(W1.at[iu1.at[c]], ub1, sem),
            pltpu.make_async_copy(W2.at[iu2.at[c]], ub2, sem),
            pltpu.make_async_copy(W0.at[ip0.at[c]], pb0, sem),
            pltpu.make_async_copy(W1.at[ip1.at[c]], pb1, sem),
            pltpu.make_async_copy(W2.at[ip2.at[c]], pb2, sem),
        ]
        for j in range(VN_DMAS):
            r = c * VN_DMAS + j
            d = pl.ds(j * IVW, IVW)
            ds.append(pltpu.make_async_copy(W0.at[iv0.at[r]], nb0.at[d], sem))
            ds.append(pltpu.make_async_copy(W1.at[iv1.at[r]], nb1.at[d], sem))
            ds.append(pltpu.make_async_copy(W2.at[iv2.at[r]], nb2.at[d], sem))
        return ds

    def fire(c, bufs, sem):
        for d in descriptors(c, bufs, sem):
            d.start()

    def drain(c, bufs, sem):
        for d in descriptors(c, bufs, sem):
            d.wait()

    def compute(c, bufs):
        ub0, ub1, ub2, pb0, pb1, pb2, nb0, nb1, nb2 = bufs

        def body(b, carry2):
            r0 = b * K
            u = ub0[b, :]
            sacc = u * pb0[b, :]
            pool = nb0[r0, :]
            for k in range(1, K):
                pool = pool + nb0[r0 + k, :]
            nacc = u * pool
            for ub, pb, nb, nj in ((ub1, pb1, nb1, 2), (ub2, pb2, nb2, 4)):
                for jj in range(nj):
                    sl = pl.ds(jj * 16, 16)
                    uu = ub[b, sl]
                    sacc = sacc + uu * pb[b, sl]
                    pool = nb[r0, sl]
                    for k in range(1, K):
                        pool = pool + nb[r0 + k, sl]
                    nacc = nacc + uu * pool
            ps[b, :] = sacc
            pn[b, :] = nacc
            return carry2

        lax.fori_loop(0, C, body, 0)
        row0 = wid * BT + c * C
        pltpu.sync_copy(ps, score_o.at[pl.ds(row0, C)])
        pltpu.sync_copy(pn, neg_o.at[pl.ds(row0, C)])

    fire(0, seta, semA)

    def step(cc, carry):
        ca = 2 * cc
        cb = 2 * cc + 1
        drain(ca, seta, semA)
        fire(cb, setb, semB)
        compute(ca, seta)

        @pl.when(cc < NCH // 2 - 1)
        def _():
            fire(ca + 2, seta, semA)

        drain(cb, setb, semB)
        compute(cb, setb)
        return carry

    lax.fori_loop(0, NCH // 2, step, 0)


def _loss_body(s_ref, n_ref, o_ref):
    s = jnp.sum(s_ref[...], axis=1)
    n = jnp.sum(n_ref[...], axis=1)

    def softplus(x):
        return jnp.maximum(x, 0.0) + jnp.log1p(jnp.exp(-jnp.abs(x)))

    part = jnp.sum(softplus(-s) + softplus(n)) * (1.0 / B)

    @pl.when(pl.program_id(0) == 0)
    def _():
        o_ref[...] = jnp.zeros_like(o_ref)

    o_ref[...] += jnp.reshape(part, (1, 1))


def _tr_body(x_ref, o_ref):
    o_ref[...] = jnp.transpose(x_ref[...])


def _transpose_tables(W1, W2):
    def tr(W, d, n, blk):
        grid = (n + blk - 1) // blk
        return pl.pallas_call(
            _tr_body,
            grid=(grid,),
            in_specs=[pl.BlockSpec((d, blk), lambda g: (0, g))],
            out_specs=pl.BlockSpec((blk, d), lambda g: (g, 0)),
            out_shape=jax.ShapeDtypeStruct((n, d), jnp.float32),
        )(jnp.transpose(W))

    return tr(W1, 32, 100000, 8192), tr(W2, 64, 1000000, 4096)


def kernel(u0, u1, u2, vp0, vp1, vp2, vn0, vn1, vn2, W0, W1, W2):
    W1p, W2p = _transpose_tables(W1, W2)
    u0r = u0.reshape(B // C, C)
    u1r = u1.reshape(B // C, C)
    u2r = u2.reshape(B // C, C)
    p0r = vp0.reshape(B // C, C)
    p1r = vp1.reshape(B // C, C)
    p2r = vp2.reshape(B // C, C)
    v0r = vn0.reshape(B * K // IVW, IVW)
    v1r = vn1.reshape(B * K // IVW, IVW)
    v2r = vn2.reshape(B * K // IVW, IVW)
    score_p, neg_p = _sc_scores(W0, W1p, W2p, u0r, u1r, u2r,
                                p0r, p1r, p2r, v0r, v1r, v2r)
    out = pl.pallas_call(
        _loss_body,
        grid=(8,),
        in_specs=[
            pl.BlockSpec((B // 8, 16), lambda i: (i, 0)),
            pl.BlockSpec((B // 8, 16), lambda i: (i, 0)),
        ],
        out_specs=pl.BlockSpec((1, 1), lambda i: (0, 0)),
        out_shape=jax.ShapeDtypeStruct((1, 1), jnp.float32),
    )(score_p, neg_p)
    return out[0, 0]


# speedup vs baseline: 1.0746x; 1.0746x over previous
import functools

import jax
import jax.numpy as jnp
from jax import lax
from jax.experimental import pallas as pl
from jax.experimental.pallas import tpu as pltpu
from jax.experimental.pallas import tpu_sc as plsc

B = 16384
K = 20
NC = 2
NS = 16
NW = NC * NS
BT = B // NW
C = 16
NCH = BT // C
VN_ROWS = C * K
IVW = 64
VN_DMAS = VN_ROWS // IVW

_mesh = plsc.VectorSubcoreMesh(core_axis_name="c", subcore_axis_name="s")

_buf_set = lambda: [
    pltpu.VMEM((C, 16), jnp.float32),
    pltpu.VMEM((C, 32), jnp.float32),
    pltpu.VMEM((C, 64), jnp.float32),
    pltpu.VMEM((C, 16), jnp.float32),
    pltpu.VMEM((C, 32), jnp.float32),
    pltpu.VMEM((C, 64), jnp.float32),
    pltpu.VMEM((VN_ROWS, 16), jnp.float32),
    pltpu.VMEM((VN_ROWS, 32), jnp.float32),
    pltpu.VMEM((VN_ROWS, 64), jnp.float32),
]


@functools.partial(
    pl.kernel,
    mesh=_mesh,
    compiler_params=pltpu.CompilerParams(use_tc_tiling_on_sc=False),
    out_type=[
        jax.ShapeDtypeStruct((B, 16), jnp.float32),
        jax.ShapeDtypeStruct((B, 16), jnp.float32),
    ],
    scratch_types=[
        pltpu.VMEM((NCH, C), jnp.int32),
        pltpu.VMEM((NCH, C), jnp.int32),
        pltpu.VMEM((NCH, C), jnp.int32),
        pltpu.VMEM((NCH, C), jnp.int32),
        pltpu.VMEM((NCH, C), jnp.int32),
        pltpu.VMEM((NCH, C), jnp.int32),
        pltpu.VMEM((NCH * VN_DMAS, IVW), jnp.int32),
        pltpu.VMEM((NCH * VN_DMAS, IVW), jnp.int32),
        pltpu.VMEM((NCH * VN_DMAS, IVW), jnp.int32),
        *_buf_set(),
        *_buf_set(),
        pltpu.VMEM((C, 16), jnp.float32),
        pltpu.VMEM((C, 16), jnp.float32),
        pltpu.SemaphoreType.DMA,
        pltpu.SemaphoreType.DMA,
    ],
)
def _sc_scores(W0, W1, W2, u0, u1, u2, p0, p1, p2, v0, v1, v2,
               score_o, neg_o,
               iu0, iu1, iu2, ip0, ip1, ip2, iv0, iv1, iv2,
               a_ub0, a_ub1, a_ub2, a_pb0, a_pb1, a_pb2, a_nb0, a_nb1, a_nb2,
               b_ub0, b_ub1, b_ub2, b_pb0, b_pb1, b_pb2, b_nb0, b_nb1, b_nb2,
               ps, pn, semA, semB):
    wid = lax.axis_index("s") * NC + lax.axis_index("c")
    seta = (a_ub0, a_ub1, a_ub2, a_pb0, a_pb1, a_pb2, a_nb0, a_nb1, a_nb2)
    setb = (b_ub0, b_ub1, b_ub2, b_pb0, b_pb1, b_pb2, b_nb0, b_nb1, b_nb2)

    hs = [
        pltpu.async_copy(u0.at[pl.ds(wid * NCH, NCH)], iu0, semA),
        pltpu.async_copy(u1.at[pl.ds(wid * NCH, NCH)], iu1, semA),
        pltpu.async_copy(u2.at[pl.ds(wid * NCH, NCH)], iu2, semA),
        pltpu.async_copy(p0.at[pl.ds(wid * NCH, NCH)], ip0, semA),
        pltpu.async_copy(p1.at[pl.ds(wid * NCH, NCH)], ip1, semA),
        pltpu.async_copy(p2.at[pl.ds(wid * NCH, NCH)], ip2, semA),
        pltpu.async_copy(v0.at[pl.ds(wid * NCH * VN_DMAS, NCH * VN_DMAS)], iv0, semA),
        pltpu.async_copy(v1.at[pl.ds(wid * NCH * VN_DMAS, NCH * VN_DMAS)], iv1, semA),
        pltpu.async_copy(v2.at[pl.ds(wid * NCH * VN_DMAS, NCH * VN_DMAS)], iv2, semA),
    ]
    for h in hs:
        h.wait()

    def descriptors(c, bufs, sem):
        ub0, ub1, ub2, pb0, pb1, pb2, nb0, nb1, nb2 = bufs
        ds = [
            pltpu.make_async_copy(W0.at[iu0.at[c]], ub0, sem),
            pltpu.make_async_copy(W1.at[iu1.at[c]], ub1, sem),
            pltpu.make_async_copy(W2.at[iu2.at[c]], ub2, sem),
            pltpu.make_async_copy(W0.at[ip0.at[c]], pb0, sem),
            pltpu.make_async_copy(W1.at[ip1.at[c]], pb1, sem),
            pltpu.make_async_copy(W2.at[ip2.at[c]], pb2, sem),
        ]
        for j in range(VN_DMAS):
            r = c * VN_DMAS + j
            d = pl.ds(j * IVW, IVW)
            ds.append(pltpu.make_async_copy(W0.at[iv0.at[r]], nb0.at[d], sem))
            ds.append(pltpu.make_async_copy(W1.at[iv1.at[r]], nb1.at[d], sem))
            ds.append(pltpu.make_async_copy(W2.at[iv2.at[r]], nb2.at[d], sem))
        return ds

    def fire(c, bufs, sem):
        for d in descriptors(c, bufs, sem):
            d.start()

    def drain(c, bufs, sem):
        for d in descriptors(c, bufs, sem):
            d.wait()

    def compute(c, bufs):
        ub0, ub1, ub2, pb0, pb1, pb2, nb0, nb1, nb2 = bufs

        def body(b, carry2):
            r0 = b * K
            u = ub0[b, :]
            sacc = u * pb0[b, :]
            pool = nb0[r0, :]
            for k in range(1, K):
                pool = pool + nb0[r0 + k, :]
            nacc = u * pool
            for ub, pb, nb, nj in ((ub1, pb1, nb1, 2), (ub2, pb2, nb2, 4)):
                for jj in range(nj):
                    sl = pl.ds(jj * 16, 16)
                    uu = ub[b, sl]
                    sacc = sacc + uu * pb[b, sl]
                    pool = nb[r0, sl]
                    for k in range(1, K):
                        pool = pool + nb[r0 + k, sl]
                    nacc = nacc + uu * pool
            ps[b, :] = sacc
            pn[b, :] = nacc
            return carry2

        lax.fori_loop(0, C, body, 0)
        row0 = wid * BT + c * C
        pltpu.sync_copy(ps, score_o.at[pl.ds(row0, C)])
        pltpu.sync_copy(pn, neg_o.at[pl.ds(row0, C)])

    fire(0, seta, semA)

    def step(cc, carry):
        ca = 2 * cc
        cb = 2 * cc + 1
        drain(ca, seta, semA)
        fire(cb, setb, semB)
        compute(ca, seta)

        @pl.when(cc < NCH // 2 - 1)
        def _():
            fire(ca + 2, seta, semA)

        drain(cb, setb, semB)
        compute(cb, setb)
        return carry

    lax.fori_loop(0, NCH // 2, step, 0)


def _loss_body(s_ref, n_ref, o_ref):
    s = jnp.sum(s_ref[...], axis=1)
    n = jnp.sum(n_ref[...], axis=1)

    def softplus(x):
        return jnp.maximum(x, 0.0) + jnp.log1p(jnp.exp(-jnp.abs(x)))

    part = jnp.sum(softplus(-s) + softplus(n)) * (1.0 / B)

    @pl.when(pl.program_id(0) == 0)
    def _():
        o_ref[...] = jnp.zeros_like(o_ref)

    o_ref[...] += jnp.reshape(part, (1, 1))


def _tr_body(d, x_ref, o_ref):
    ii = lax.broadcasted_iota(jnp.int32, (d, d), 0)
    jj = lax.broadcasted_iota(jnp.int32, (d, d), 1)
    eye = (ii == jj).astype(jnp.float32)
    o_ref[...] = lax.dot_general(
        x_ref[...], eye, (((0,), (0,)), ((), ())),
        preferred_element_type=jnp.float32)


def _transpose_tables(W1, W2):
    def tr(W, d, n, blk):
        grid = (n + blk - 1) // blk
        return pl.pallas_call(
            functools.partial(_tr_body, d),
            grid=(grid,),
            in_specs=[pl.BlockSpec((d, blk), lambda g: (0, g))],
            out_specs=pl.BlockSpec((blk, d), lambda g: (g, 0)),
            out_shape=jax.ShapeDtypeStruct((n, d), jnp.float32),
        )(jnp.transpose(W))

    return tr(W1, 32, 100000, 8192), tr(W2, 64, 1000000, 8192)


def kernel(u0, u1, u2, vp0, vp1, vp2, vn0, vn1, vn2, W0, W1, W2):
    W1p, W2p = _transpose_tables(W1, W2)
    u0r = u0.reshape(B // C, C)
    u1r = u1.reshape(B // C, C)
    u2r = u2.reshape(B // C, C)
    p0r = vp0.reshape(B // C, C)
    p1r = vp1.reshape(B // C, C)
    p2r = vp2.reshape(B // C, C)
    v0r = vn0.reshape(B * K // IVW, IVW)
    v1r = vn1.reshape(B * K // IVW, IVW)
    v2r = vn2.reshape(B * K // IVW, IVW)
    score_p, neg_p = _sc_scores(W0, W1p, W2p, u0r, u1r, u2r,
                                p0r, p1r, p2r, v0r, v1r, v2r)
    out = pl.pallas_call(
        _loss_body,
        grid=(8,),
        in_specs=[
            pl.BlockSpec((B // 8, 16), lambda i: (i, 0)),
            pl.BlockSpec((B // 8, 16), lambda i: (i, 0)),
        ],
        out_specs=pl.BlockSpec((1, 1), lambda i: (0, 0)),
        out_shape=jax.ShapeDtypeStruct((1, 1), jnp.float32),
    )(score_p, neg_p)
    return out[0, 0]


# speedup vs baseline: 1.2316x; 1.1460x over previous
import functools

import jax
import jax.numpy as jnp
from jax import lax
from jax.experimental import pallas as pl
from jax.experimental.pallas import tpu as pltpu
from jax.experimental.pallas import tpu_sc as plsc

B = 16384
K = 20
NC = 2
NS = 16
NW = NC * NS
BT = B // NW
C = 16
NCH = BT // C
VN_ROWS = C * K
IVW = 64
VN_DMAS = VN_ROWS // IVW

_mesh = plsc.VectorSubcoreMesh(core_axis_name="c", subcore_axis_name="s")

_buf_set = lambda: [
    pltpu.VMEM((C, 16), jnp.float32),
    pltpu.VMEM((C, 32), jnp.float32),
    pltpu.VMEM((C, 64), jnp.float32),
    pltpu.VMEM((C, 16), jnp.float32),
    pltpu.VMEM((C, 32), jnp.float32),
    pltpu.VMEM((C, 64), jnp.float32),
    pltpu.VMEM((VN_ROWS, 16), jnp.float32),
    pltpu.VMEM((VN_ROWS, 32), jnp.float32),
    pltpu.VMEM((VN_ROWS, 64), jnp.float32),
]


@functools.partial(
    pl.kernel,
    mesh=_mesh,
    compiler_params=pltpu.CompilerParams(use_tc_tiling_on_sc=False),
    out_type=[
        jax.ShapeDtypeStruct((B, 16), jnp.float32),
        jax.ShapeDtypeStruct((B, 16), jnp.float32),
    ],
    scratch_types=[
        pltpu.VMEM((NCH, C), jnp.int32),
        pltpu.VMEM((NCH, C), jnp.int32),
        pltpu.VMEM((NCH, C), jnp.int32),
        pltpu.VMEM((NCH, C), jnp.int32),
        pltpu.VMEM((NCH, C), jnp.int32),
        pltpu.VMEM((NCH, C), jnp.int32),
        pltpu.VMEM((NCH * VN_DMAS, IVW), jnp.int32),
        pltpu.VMEM((NCH * VN_DMAS, IVW), jnp.int32),
        pltpu.VMEM((NCH * VN_DMAS, IVW), jnp.int32),
        *_buf_set(),
        *_buf_set(),
        pltpu.VMEM((C, 16), jnp.float32),
        pltpu.VMEM((C, 16), jnp.float32),
        pltpu.SemaphoreType.DMA,
        pltpu.SemaphoreType.DMA,
    ],
)
def _sc_scores(W0, W1, W2, u0, u1, u2, p0, p1, p2, v0, v1, v2,
               score_o, neg_o,
               iu0, iu1, iu2, ip0, ip1, ip2, iv0, iv1, iv2,
               a_ub0, a_ub1, a_ub2, a_pb0, a_pb1, a_pb2, a_nb0, a_nb1, a_nb2,
               b_ub0, b_ub1, b_ub2, b_pb0, b_pb1, b_pb2, b_nb0, b_nb1, b_nb2,
               ps, pn, semA, semB):
    wid = lax.axis_index("s") * NC + lax.axis_index("c")
    seta = (a_ub0, a_ub1, a_ub2, a_pb0, a_pb1, a_pb2, a_nb0, a_nb1, a_nb2)
    setb = (b_ub0, b_ub1, b_ub2, b_pb0, b_pb1, b_pb2, b_nb0, b_nb1, b_nb2)

    hs = [
        pltpu.async_copy(u0.at[pl.ds(wid * NCH, NCH)], iu0, semA),
        pltpu.async_copy(u1.at[pl.ds(wid * NCH, NCH)], iu1, semA),
        pltpu.async_copy(u2.at[pl.ds(wid * NCH, NCH)], iu2, semA),
        pltpu.async_copy(p0.at[pl.ds(wid * NCH, NCH)], ip0, semA),
        pltpu.async_copy(p1.at[pl.ds(wid * NCH, NCH)], ip1, semA),
        pltpu.async_copy(p2.at[pl.ds(wid * NCH, NCH)], ip2, semA),
        pltpu.async_copy(v0.at[pl.ds(wid * NCH * VN_DMAS, NCH * VN_DMAS)], iv0, semA),
        pltpu.async_copy(v1.at[pl.ds(wid * NCH * VN_DMAS, NCH * VN_DMAS)], iv1, semA),
        pltpu.async_copy(v2.at[pl.ds(wid * NCH * VN_DMAS, NCH * VN_DMAS)], iv2, semA),
    ]
    for h in hs:
        h.wait()

    def descriptors(c, bufs, sem):
        ub0, ub1, ub2, pb0, pb1, pb2, nb0, nb1, nb2 = bufs
        ds = [
            pltpu.make_async_copy(W0.at[iu0.at[c]], ub0, sem),
            pltpu.make_async_copy(W1.at[iu1.at[c]], ub1, sem),
            pltpu.make_async_copy(W2.at[iu2.at[c]], ub2, sem),
            pltpu.make_async_copy(W0.at[ip0.at[c]], pb0, sem),
            pltpu.make_async_copy(W1.at[ip1.at[c]], pb1, sem),
            pltpu.make_async_copy(W2.at[ip2.at[c]], pb2, sem),
        ]
        for j in range(VN_DMAS):
            r = c * VN_DMAS + j
            d = pl.ds(j * IVW, IVW)
            ds.append(pltpu.make_async_copy(W0.at[iv0.at[r]], nb0.at[d], sem))
            ds.append(pltpu.make_async_copy(W1.at[iv1.at[r]], nb1.at[d], sem))
            ds.append(pltpu.make_async_copy(W2.at[iv2.at[r]], nb2.at[d], sem))
        return ds

    def fire(c, bufs, sem):
        for d in descriptors(c, bufs, sem):
            d.start()

    def drain(c, bufs, sem):
        for d in descriptors(c, bufs, sem):
            d.wait()

    def compute(c, bufs):
        ub0, ub1, ub2, pb0, pb1, pb2, nb0, nb1, nb2 = bufs

        def body(b, carry2):
            r0 = b * K
            u = ub0[b, :]
            sacc = u * pb0[b, :]
            pool = nb0[r0, :]
            for k in range(1, K):
                pool = pool + nb0[r0 + k, :]
            nacc = u * pool
            for ub, pb, nb, nj in ((ub1, pb1, nb1, 2), (ub2, pb2, nb2, 4)):
                for jj in range(nj):
                    sl = pl.ds(jj * 16, 16)
                    uu = ub[b, sl]
                    sacc = sacc + uu * pb[b, sl]
                    pool = nb[r0, sl]
                    for k in range(1, K):
                        pool = pool + nb[r0 + k, sl]
                    nacc = nacc + uu * pool
            ps[b, :] = sacc
            pn[b, :] = nacc
            return carry2

        lax.fori_loop(0, C, body, 0)
        row0 = wid * BT + c * C
        pltpu.sync_copy(ps, score_o.at[pl.ds(row0, C)])
        pltpu.sync_copy(pn, neg_o.at[pl.ds(row0, C)])

    fire(0, seta, semA)

    def step(cc, carry):
        ca = 2 * cc
        cb = 2 * cc + 1
        drain(ca, seta, semA)
        fire(cb, setb, semB)
        compute(ca, seta)

        @pl.when(cc < NCH // 2 - 1)
        def _():
            fire(ca + 2, seta, semA)

        drain(cb, setb, semB)
        compute(cb, setb)
        return carry

    lax.fori_loop(0, NCH // 2, step, 0)


def _loss_body(s_ref, n_ref, o_ref):
    s = jnp.sum(s_ref[...], axis=1)
    n = jnp.sum(n_ref[...], axis=1)

    def softplus(x):
        return jnp.maximum(x, 0.0) + jnp.log1p(jnp.exp(-jnp.abs(x)))

    part = jnp.sum(softplus(-s) + softplus(n)) * (1.0 / B)

    @pl.when(pl.program_id(0) == 0)
    def _():
        o_ref[...] = jnp.zeros_like(o_ref)

    o_ref[...] += jnp.reshape(part, (1, 1))


def kernel(u0, u1, u2, vp0, vp1, vp2, vn0, vn1, vn2, W0, W1, W2):
    u0r = u0.reshape(B // C, C)
    u1r = u1.reshape(B // C, C)
    u2r = u2.reshape(B // C, C)
    p0r = vp0.reshape(B // C, C)
    p1r = vp1.reshape(B // C, C)
    p2r = vp2.reshape(B // C, C)
    v0r = vn0.reshape(B * K // IVW, IVW)
    v1r = vn1.reshape(B * K // IVW, IVW)
    v2r = vn2.reshape(B * K // IVW, IVW)
    score_p, neg_p = _sc_scores(W0, W1, W2, u0r, u1r, u2r,
                                p0r, p1r, p2r, v0r, v1r, v2r)
    out = pl.pallas_call(
        _loss_body,
        grid=(8,),
        in_specs=[
            pl.BlockSpec((B // 8, 16), lambda i: (i, 0)),
            pl.BlockSpec((B // 8, 16), lambda i: (i, 0)),
        ],
        out_specs=pl.BlockSpec((1, 1), lambda i: (0, 0)),
        out_shape=jax.ShapeDtypeStruct((1, 1), jnp.float32),
    )(score_p, neg_p)
    return out[0, 0]
